# Initial kernel scaffold; baseline (speedup 1.0000x reference)
import jax
import jax.numpy as jnp
from jax.experimental import pallas as pl


def kernel(tensor, int_to_feat_matrix, extra_embeddings):
    raise NotImplementedError("write your pallas kernel here")


# speedup vs baseline: 15.7631x; 15.7631x over previous
import functools

import jax
import jax.numpy as jnp
from jax import lax
from jax.experimental import pallas as pl
from jax.experimental.pallas import tpu as pltpu
from jax.experimental.pallas import tpu_sc as plsc

_MAX = 255
_D = 20
_NC, _NS, _L = 2, 16, 16
_NW = _NC * _NS
_RB = 16


@jax.jit
def _gather(table, tensor_i32):
    rows, feat = tensor_i32.shape
    rows_w = rows // _NW
    nblk = rows_w // _RB
    assert nblk % 2 == 0 and nblk >= 4
    mesh = plsc.VectorSubcoreMesh(core_axis_name="c", subcore_axis_name="s")

    @functools.partial(
        pl.kernel,
        out_type=jax.ShapeDtypeStruct((rows, feat * _D), jnp.float32),
        mesh=mesh,
        compiler_params=pltpu.CompilerParams(needs_layout_passes=False),
        scratch_types=[
            pltpu.VMEM(((_MAX + 1) * _D,), jnp.float32),
            pltpu.VMEM((_RB, feat), jnp.int32),
            pltpu.VMEM((_RB, feat), jnp.int32),
            pltpu.VMEM((_RB, feat * _D), jnp.float32),
            pltpu.VMEM((_RB, feat * _D), jnp.float32),
            pltpu.SemaphoreType.DMA,
            pltpu.SemaphoreType.DMA,
            pltpu.SemaphoreType.DMA,
            pltpu.SemaphoreType.DMA,
        ],
    )
    def k(table_hbm, t_hbm, out_hbm, table_v,
          idx0, idx1, outv0, outv1, isem0, isem1, osem0, osem1):
        wid = lax.axis_index("s") * _NC + lax.axis_index("c")
        pltpu.sync_copy(table_hbm, table_v)
        lanes = lax.iota(jnp.int32, _L)
        row0 = wid * rows_w

        idx_bufs = (idx0, idx1)
        out_bufs = (outv0, outv1)
        isems = (isem0, isem1)
        osems = (osem0, osem1)

        pltpu.async_copy(t_hbm.at[pl.ds(row0, _RB), :], idx0, isem0)
        pltpu.async_copy(t_hbm.at[pl.ds(row0 + _RB, _RB), :], idx1, isem1)

        def half(j, t):
            b = j * 2 + t
            rbase = row0 + b * _RB
            idx_v, out_v = idx_bufs[t], out_bufs[t]
            isem, osem = isems[t], osems[t]
            pltpu.make_async_copy(
                t_hbm.at[pl.ds(rbase, _RB), :], idx_v, isem).wait()

            @pl.when(j > 0)
            def _():
                pltpu.make_async_copy(
                    out_v, out_hbm.at[pl.ds(rbase, _RB), :], osem).wait()

            @plsc.parallel_loop(0, feat)
            def _grp(f):
                fv = jnp.full((_L,), f, jnp.int32)
                iv = plsc.load_gather(idx_v, [lanes, fv])
                iv = jnp.minimum(jnp.maximum(iv, 0), _MAX) * _D
                colb = fv * _D
                vals = [plsc.load_gather(table_v, [iv + d]) for d in range(_D)]
                for d in range(_D):
                    plsc.store_scatter(out_v, [lanes, colb + d], vals[d])

            pltpu.async_copy(out_v, out_hbm.at[pl.ds(rbase, _RB), :], osem)

            @pl.when(b + 2 < nblk)
            def _():
                pltpu.async_copy(
                    t_hbm.at[pl.ds(rbase + 2 * _RB, _RB), :], idx_v, isem)

        def blk2(j, carry):
            half(j, 0)
            half(j, 1)
            return carry

        lax.fori_loop(0, nblk // 2, blk2, 0)
        pltpu.make_async_copy(
            outv0,
            out_hbm.at[pl.ds(row0 + (nblk - 2) * _RB, _RB), :], osem0).wait()
        pltpu.make_async_copy(
            outv1,
            out_hbm.at[pl.ds(row0 + (nblk - 1) * _RB, _RB), :], osem1).wait()

    return k(table, tensor_i32)


def kernel(tensor, int_to_feat_matrix, extra_embeddings):
    orig_shape = tensor.shape
    t2 = tensor.reshape(-1, orig_shape[-1]).astype(jnp.int32)
    table = jnp.concatenate(
        [int_to_feat_matrix, extra_embeddings[:1]], axis=0).reshape(-1)
    out = _gather(table, t2)
    return out.reshape(*orig_shape[:-1], orig_shape[-1] * _D)


# speedup vs baseline: 21.4643x; 1.3617x over previous
import functools

import jax
import jax.numpy as jnp
from jax import lax
from jax.experimental import pallas as pl
from jax.experimental.pallas import tpu as pltpu
from jax.experimental.pallas import tpu_sc as plsc

_MAX = 255
_D = 20
_NC, _NS, _L = 2, 16, 16
_NW = _NC * _NS
_RB = 16


@jax.jit
def _gather(table, tensor_i32):
    rows, feat = tensor_i32.shape
    rows_w = rows // _NW
    nblk = rows_w // _RB
    assert nblk % 2 == 0 and nblk >= 4
    mesh = plsc.VectorSubcoreMesh(core_axis_name="c", subcore_axis_name="s")

    @functools.partial(
        pl.kernel,
        out_type=jax.ShapeDtypeStruct((rows, feat * _D), jnp.float32),
        mesh=mesh,
        compiler_params=pltpu.CompilerParams(needs_layout_passes=False),
        scratch_types=[
            pltpu.VMEM(((_MAX + 1) * _D,), jnp.float32),
            pltpu.VMEM((_RB, feat), jnp.int32),
            pltpu.VMEM((_RB, feat), jnp.int32),
            pltpu.VMEM((_RB, feat * _D), jnp.float32),
            pltpu.VMEM((_RB, feat * _D), jnp.float32),
            pltpu.SemaphoreType.DMA,
            pltpu.SemaphoreType.DMA,
            pltpu.SemaphoreType.DMA,
            pltpu.SemaphoreType.DMA,
        ],
    )
    def k(table_hbm, t_hbm, out_hbm, table_v,
          idx0, idx1, outv0, outv1, isem0, isem1, osem0, osem1):
        wid = lax.axis_index("s") * _NC + lax.axis_index("c")
        pltpu.sync_copy(table_hbm, table_v)
        lanes = lax.iota(jnp.int32, _L)
        row0 = wid * rows_w

        idx_bufs = (idx0, idx1)
        out_bufs = (outv0, outv1)
        isems = (isem0, isem1)
        osems = (osem0, osem1)

        pltpu.async_copy(t_hbm.at[pl.ds(row0, _RB), :], idx0, isem0)
        pltpu.async_copy(t_hbm.at[pl.ds(row0 + _RB, _RB), :], idx1, isem1)

        def half(j, t):
            b = j * 2 + t
            rbase = row0 + b * _RB
            idx_v, out_v = idx_bufs[t], out_bufs[t]
            isem, osem = isems[t], osems[t]
            pltpu.make_async_copy(
                t_hbm.at[pl.ds(rbase, _RB), :], idx_v, isem).wait()

            @pl.when(j > 0)
            def _():
                pltpu.make_async_copy(
                    out_v, out_hbm.at[pl.ds(rbase, _RB), :], osem).wait()

            dvec = lax.iota(jnp.int32, _L)
            starts = list(range(0, feat - _L + 1, _L))
            if starts[-1] != feat - _L:
                starts.append(feat - _L)

            @plsc.parallel_loop(0, _RB)
            def _row(r):
                rv = jnp.full((_L,), r, jnp.int32)
                for s in starts:
                    ivv = idx_v[r, pl.ds(s, _L)]
                    ivv = jnp.minimum(jnp.maximum(ivv, 0), _MAX) * _D
                    for j in range(_L):
                        ivb = ivv[jnp.full((_L,), j, jnp.int32)]
                        a0 = ivb + dvec
                        v0 = plsc.load_gather(table_v, [a0])
                        v1 = plsc.load_gather(table_v, [a0 + (_D - _L)])
                        colv = dvec + (s + j) * _D
                        plsc.store_scatter(out_v, [rv, colv], v0)
                        plsc.store_scatter(
                            out_v, [rv, colv + (_D - _L)], v1)

            pltpu.async_copy(out_v, out_hbm.at[pl.ds(rbase, _RB), :], osem)

            @pl.when(b + 2 < nblk)
            def _():
                pltpu.async_copy(
                    t_hbm.at[pl.ds(rbase + 2 * _RB, _RB), :], idx_v, isem)

        def blk2(j, carry):
            half(j, 0)
            half(j, 1)
            return carry

        lax.fori_loop(0, nblk // 2, blk2, 0)
        pltpu.make_async_copy(
            outv0,
            out_hbm.at[pl.ds(row0 + (nblk - 2) * _RB, _RB), :], osem0).wait()
        pltpu.make_async_copy(
            outv1,
            out_hbm.at[pl.ds(row0 + (nblk - 1) * _RB, _RB), :], osem1).wait()

    return k(table, tensor_i32)


def kernel(tensor, int_to_feat_matrix, extra_embeddings):
    orig_shape = tensor.shape
    t2 = tensor.reshape(-1, orig_shape[-1]).astype(jnp.int32)
    table = jnp.concatenate(
        [int_to_feat_matrix, extra_embeddings[:1]], axis=0).reshape(-1)
    out = _gather(table, t2)
    return out.reshape(*orig_shape[:-1], orig_shape[-1] * _D)
